# Initial kernel scaffold; baseline (speedup 1.0000x reference)
import jax
import jax.numpy as jnp
from jax.experimental import pallas as pl


def kernel(seq, adj, W, bias, prelu_a):
    raise NotImplementedError("write your pallas kernel here")


# speedup vs baseline: 1.0385x; 1.0385x over previous
import jax
import jax.numpy as jnp
from jax.experimental import pallas as pl
from jax.experimental.pallas import tpu as pltpu


def _gcn_body(seq_ref, w_ref, a_ref, adj_ref, bias_ref, out_ref, fts_ref):
    @pl.when(pl.program_id(0) == 0)
    def _project():
        fts_ref[...] = jax.lax.dot_general(
            seq_ref[...], w_ref[...],
            dimension_numbers=(((1,), (1,)), ((), ())),
            preferred_element_type=jnp.float32)

    acc = jax.lax.dot_general(
        adj_ref[...], fts_ref[...],
        dimension_numbers=(((1,), (0,)), ((), ())),
        preferred_element_type=jnp.float32)
    acc = acc + bias_ref[...]
    a = a_ref[0]
    out_ref[...] = jnp.where(acc >= 0, acc, a * acc)


def _block_m(n: int) -> int:
    best = 8
    for bm in range(8, 513, 8):
        if n % bm == 0:
            best = bm
    return best


def kernel(seq, adj, W, bias, prelu_a):
    b, n, d_in = seq.shape
    d_out = W.shape[0]
    seq2 = seq.reshape(b * n, d_in)
    adj2 = adj.reshape(b * n, n)
    bias2 = bias.reshape(1, d_out)
    a2 = jnp.asarray(prelu_a, jnp.float32).reshape(1)

    bm = _block_m(b * n)
    grid = (b * n // bm,)

    out = pl.pallas_call(
        _gcn_body,
        grid=grid,
        in_specs=[
            pl.BlockSpec((b * n, d_in), lambda i: (0, 0)),
            pl.BlockSpec((d_out, d_in), lambda i: (0, 0)),
            pl.BlockSpec(memory_space=pltpu.SMEM),
            pl.BlockSpec((bm, n), lambda i: (i, 0)),
            pl.BlockSpec((1, d_out), lambda i: (0, 0)),
        ],
        out_specs=pl.BlockSpec((bm, d_out), lambda i: (i, 0)),
        out_shape=jax.ShapeDtypeStruct((b * n, d_out), jnp.float32),
        scratch_shapes=[pltpu.VMEM((b * n, d_out), jnp.float32)],
    )(seq2, W, a2, adj2, bias2)
    return out.reshape(b, n, d_out)
